# Initial kernel scaffold; baseline (speedup 1.0000x reference)
import jax
import jax.numpy as jnp
from jax.experimental import pallas as pl


def kernel(x, edge_index, W1, b1, W2, b2, temp):
    raise NotImplementedError("write your pallas kernel here")


# speedup vs baseline: 5.7809x; 5.7809x over previous
import functools

import jax
import jax.numpy as jnp
from jax import lax
from jax.experimental import pallas as pl
from jax.experimental.pallas import tpu as pltpu
from jax.experimental.pallas import tpu_sc as plsc

N = 10000
E = 320000
F = 128
HID = 64
C = 40
K = 10
CP = 64
NW = 32
N2 = 2 * N
NV = N2 // 16
CE = 4000
NCHUNK = E // CE


def _mlp_body(x_ref, w1_ref, b1_ref, w2_ref, b2_ref, o_ref):
    h1 = jnp.maximum(jnp.dot(x_ref[...], w1_ref[...],
                             preferred_element_type=jnp.float32) + b1_ref[...], 0.0)
    o_ref[...] = jnp.dot(h1, w2_ref[...],
                         preferred_element_type=jnp.float32) + b2_ref[...]


def _lsm_body(u_ref, o_ref):
    u = u_ref[...]
    ids = lax.broadcasted_iota(jnp.int32, u.shape, 1)
    m = jnp.where(ids < C, u, jnp.float32(-1e30))
    mx = jnp.max(m, axis=1, keepdims=True)
    e = jnp.exp(m - mx)
    s = jnp.sum(e, axis=1, keepdims=True)
    o_ref[...] = (u - mx - jnp.log(s))[:, :C]


def _sc_body(h_hbm, row_hbm, col_hbm, tb_hbm, out_hbm, Vv, Tv, Hqv, dsv, rbuf, cbuf, tv):
    wid = lax.axis_index("s") * 2 + lax.axis_index("c")
    ones = jnp.ones((16,), jnp.float32)

    pltpu.sync_copy(tb_hbm, tv)
    pltpu.sync_copy(h_hbm.at[wid], Vv)

    def _init_t(i, c):
        Tv[pl.ds(i * 16, 16)] = ones
        return c
    lax.fori_loop(0, NV, _init_t, 0)

    def _deg_chunk(ch, c):
        pltpu.sync_copy(col_hbm.at[pl.ds(ch * CE, CE)], cbuf)

        def _grp(g, cc):
            cv = cbuf[pl.ds(g * 16, 16)]
            c2 = cv + cv
            plsc.addupdate_scatter(Tv, [c2], ones)
            plsc.addupdate_scatter(Tv, [c2 + 1], ones)
            return cc
        lax.fori_loop(0, CE // 16, _grp, 0)
        return c
    lax.fori_loop(0, NCHUNK, _deg_chunk, 0)

    t0 = tv[pl.ds(0, 16)]

    def _node_init(i, c):
        sl = pl.ds(i * 16, 16)
        d = Tv[sl]
        di = plsc.bitcast(d, jnp.int32)
        y = plsc.bitcast(jnp.int32(0x5F3759DF) - lax.shift_right_logical(di, 1),
                         jnp.float32)
        for _ in range(4):
            y = y * (jnp.float32(1.5) - jnp.float32(0.5) * d * y * y)
        dsv[sl] = y
        q = y * Vv[sl]
        Vv[sl] = q
        Hqv[sl] = t0 * q
        return c
    lax.fori_loop(0, NV, _node_init, 0)

    for k in range(K):
        def _self_init(i, c):
            sl = pl.ds(i * 16, 16)
            Tv[sl] = Vv[sl]
            return c
        lax.fori_loop(0, NV, _self_init, 0)

        def _edge_chunk(ch, c):
            pltpu.sync_copy(row_hbm.at[pl.ds(ch * CE, CE)], rbuf)
            pltpu.sync_copy(col_hbm.at[pl.ds(ch * CE, CE)], cbuf)

            def _grp(g, cc):
                sl = pl.ds(g * 16, 16)
                rv = rbuf[sl]
                cv = cbuf[sl]
                r2 = rv + rv
                c2 = cv + cv
                g0 = plsc.load_gather(Vv, [r2])
                g1 = plsc.load_gather(Vv, [r2 + 1])
                plsc.addupdate_scatter(Tv, [c2], g0)
                plsc.addupdate_scatter(Tv, [c2 + 1], g1)
                return cc
            lax.fori_loop(0, CE // 16, _grp, 0)
            return c
        lax.fori_loop(0, NCHUNK, _edge_chunk, 0)

        tk = tv[pl.ds(16 * (k + 1), 16)]

        def _scale(i, c):
            sl = pl.ds(i * 16, 16)
            s = dsv[sl]
            q = s * s * Tv[sl]
            Vv[sl] = q
            Hqv[sl] = Hqv[sl] + tk * q
            return c
        lax.fori_loop(0, NV, _scale, 0)

    def _fin(i, c):
        sl = pl.ds(i * 16, 16)
        Vv[sl] = Hqv[sl] / dsv[sl]
        return c
    lax.fori_loop(0, NV, _fin, 0)
    pltpu.sync_copy(Vv, out_hbm.at[wid])


_sc_prop = functools.partial(
    pl.kernel,
    out_type=jax.ShapeDtypeStruct((NW, N2), jnp.float32),
    mesh=plsc.VectorSubcoreMesh(core_axis_name="c", subcore_axis_name="s"),
    compiler_params=pltpu.CompilerParams(needs_layout_passes=False),
    scratch_types=[
        pltpu.VMEM((N2,), jnp.float32),
        pltpu.VMEM((N2,), jnp.float32),
        pltpu.VMEM((N2,), jnp.float32),
        pltpu.VMEM((N2,), jnp.float32),
        pltpu.VMEM((CE,), jnp.int32),
        pltpu.VMEM((CE,), jnp.int32),
        pltpu.VMEM((16 * (K + 1),), jnp.float32),
    ],
)(_sc_body)


def kernel(x, edge_index, W1, b1, W2, b2, temp):
    w2p = jnp.pad(W2, ((0, 0), (0, CP - C)))
    b2p = jnp.pad(b2, (0, CP - C)).reshape(1, CP)
    h_pad = pl.pallas_call(
        _mlp_body,
        out_shape=jax.ShapeDtypeStruct((N, CP), jnp.float32),
    )(x, W1, b1.reshape(1, HID), w2p, b2p)

    h32 = h_pad.reshape(N, NW, 2).transpose(1, 0, 2).reshape(NW, N2)
    tb = jnp.broadcast_to(temp.reshape(K + 1, 1), (K + 1, 16)).reshape(-1)
    ei = edge_index.astype(jnp.int32)
    u32 = _sc_prop(h32, ei[0], ei[1], tb)
    u = u32.reshape(NW, N, 2).transpose(1, 0, 2).reshape(N, CP)

    return pl.pallas_call(
        _lsm_body,
        out_shape=jax.ShapeDtypeStruct((N, C), jnp.float32),
    )(u)


# speedup vs baseline: 8.2773x; 1.4318x over previous
import functools

import jax
import jax.numpy as jnp
from jax import lax
from jax.experimental import pallas as pl
from jax.experimental.pallas import tpu as pltpu
from jax.experimental.pallas import tpu_sc as plsc

N = 10000
E = 320000
F = 128
HID = 64
C = 40
K = 10
CP = 64
NW = 32
N2 = 2 * N
NV = N2 // 16
CE = 8000
NCHUNK = E // CE


def _mlp_body(x_ref, w1_ref, b1_ref, w2_ref, b2_ref, o_ref):
    h1 = jnp.maximum(jnp.dot(x_ref[...], w1_ref[...],
                             preferred_element_type=jnp.float32) + b1_ref[...], 0.0)
    o_ref[...] = jnp.dot(h1, w2_ref[...],
                         preferred_element_type=jnp.float32) + b2_ref[...]


def _lsm_body(u_ref, o_ref):
    u = u_ref[...]
    ids = lax.broadcasted_iota(jnp.int32, u.shape, 1)
    m = jnp.where(ids < C, u, jnp.float32(-1e30))
    mx = jnp.max(m, axis=1, keepdims=True)
    e = jnp.exp(m - mx)
    s = jnp.sum(e, axis=1, keepdims=True)
    o_ref[...] = (u - mx - jnp.log(s))[:, :C]


def _sc_body(h_hbm, row_hbm, col_hbm, tb_hbm, out_hbm,
             Vv, Tv, Hqv, dsv, rb0, rb1, cb0, cb1, tv, sem0, sem1):
    wid = lax.axis_index("s") * 2 + lax.axis_index("c")
    ones = jnp.ones((16,), jnp.float32)
    rb = (rb0, rb1)
    cb = (cb0, cb1)
    sems = (sem0, sem1)

    pltpu.sync_copy(tb_hbm, tv)
    pltpu.sync_copy(h_hbm.at[wid], Vv)

    def _issue(ch, b, use_row):
        if use_row:
            pltpu.async_copy(row_hbm.at[pl.ds(ch * CE, CE)], rb[b], sems[b])
        pltpu.async_copy(col_hbm.at[pl.ds(ch * CE, CE)], cb[b], sems[b])

    def _wait(b, use_row):
        if use_row:
            pltpu.make_async_copy(row_hbm.at[pl.ds(0, CE)], rb[b], sems[b]).wait()
        pltpu.make_async_copy(col_hbm.at[pl.ds(0, CE)], cb[b], sems[b]).wait()

    def _edge_pass(grp_fn, use_row):
        def _process(b):
            def _grp(g, cc):
                grp_fn(rb[b], cb[b], g)
                return cc
            lax.fori_loop(0, CE // 16, _grp, 0, unroll=10)

        _issue(0, 0, use_row)

        def _pair(p, c):
            ch0 = p * 2
            _issue(ch0 + 1, 1, use_row)
            _wait(0, use_row)
            _process(0)

            @pl.when(p + 1 < NCHUNK // 2)
            def _():
                _issue(ch0 + 2, 0, use_row)
            _wait(1, use_row)
            _process(1)
            return c
        lax.fori_loop(0, NCHUNK // 2, _pair, 0)

    def _init_t(i, c):
        Tv[pl.ds(i * 16, 16)] = ones
        return c
    lax.fori_loop(0, NV, _init_t, 0, unroll=10)

    def _deg_grp(rbuf, cbuf, g):
        cv = cbuf[pl.ds(g * 16, 16)]
        c2 = cv + cv
        plsc.addupdate_scatter(Tv, [c2], ones)
        plsc.addupdate_scatter(Tv, [c2 + 1], ones)
    _edge_pass(_deg_grp, use_row=False)

    t0 = tv[pl.ds(0, 16)]

    def _node_init(i, c):
        sl = pl.ds(i * 16, 16)
        d = Tv[sl]
        di = plsc.bitcast(d, jnp.int32)
        y = plsc.bitcast(jnp.int32(0x5F3759DF) - lax.shift_right_logical(di, 1),
                         jnp.float32)
        for _ in range(4):
            y = y * (jnp.float32(1.5) - jnp.float32(0.5) * d * y * y)
        dsv[sl] = y
        q = y * Vv[sl]
        Vv[sl] = q
        Tv[sl] = q
        Hqv[sl] = t0 * q
        return c
    lax.fori_loop(0, NV, _node_init, 0, unroll=5)

    def _edge_grp(rbuf, cbuf, g):
        sl = pl.ds(g * 16, 16)
        rv = rbuf[sl]
        cv = cbuf[sl]
        r2 = rv + rv
        c2 = cv + cv
        g0 = plsc.load_gather(Vv, [r2])
        g1 = plsc.load_gather(Vv, [r2 + 1])
        plsc.addupdate_scatter(Tv, [c2], g0)
        plsc.addupdate_scatter(Tv, [c2 + 1], g1)

    for k in range(K):
        _edge_pass(_edge_grp, use_row=True)
        tk = tv[pl.ds(16 * (k + 1), 16)]

        def _scale(i, c):
            sl = pl.ds(i * 16, 16)
            s = dsv[sl]
            q = s * s * Tv[sl]
            Vv[sl] = q
            Tv[sl] = q
            Hqv[sl] = Hqv[sl] + tk * q
            return c
        lax.fori_loop(0, NV, _scale, 0, unroll=10)

    def _fin(i, c):
        sl = pl.ds(i * 16, 16)
        Vv[sl] = Hqv[sl] / dsv[sl]
        return c
    lax.fori_loop(0, NV, _fin, 0, unroll=10)
    pltpu.sync_copy(Vv, out_hbm.at[wid])


_sc_prop = functools.partial(
    pl.kernel,
    out_type=jax.ShapeDtypeStruct((NW, N2), jnp.float32),
    mesh=plsc.VectorSubcoreMesh(core_axis_name="c", subcore_axis_name="s"),
    compiler_params=pltpu.CompilerParams(needs_layout_passes=False),
    scratch_types=[
        pltpu.VMEM((N2,), jnp.float32),
        pltpu.VMEM((N2,), jnp.float32),
        pltpu.VMEM((N2,), jnp.float32),
        pltpu.VMEM((N2,), jnp.float32),
        pltpu.VMEM((CE,), jnp.int32),
        pltpu.VMEM((CE,), jnp.int32),
        pltpu.VMEM((CE,), jnp.int32),
        pltpu.VMEM((CE,), jnp.int32),
        pltpu.VMEM((16 * (K + 1),), jnp.float32),
        pltpu.SemaphoreType.DMA,
        pltpu.SemaphoreType.DMA,
    ],
)(_sc_body)


def kernel(x, edge_index, W1, b1, W2, b2, temp):
    w2p = jnp.pad(W2, ((0, 0), (0, CP - C)))
    b2p = jnp.pad(b2, (0, CP - C)).reshape(1, CP)
    h_pad = pl.pallas_call(
        _mlp_body,
        out_shape=jax.ShapeDtypeStruct((N, CP), jnp.float32),
    )(x, W1, b1.reshape(1, HID), w2p, b2p)

    h32 = h_pad.reshape(N, NW, 2).transpose(1, 0, 2).reshape(NW, N2)
    tb = jnp.broadcast_to(temp.reshape(K + 1, 1), (K + 1, 16)).reshape(-1)
    ei = edge_index.astype(jnp.int32)
    u32 = _sc_prop(h32, ei[0], ei[1], tb)
    u = u32.reshape(NW, N, 2).transpose(1, 0, 2).reshape(N, CP)

    return pl.pallas_call(
        _lsm_body,
        out_shape=jax.ShapeDtypeStruct((N, C), jnp.float32),
    )(u)


# speedup vs baseline: 9.8610x; 1.1913x over previous
import functools

import jax
import jax.numpy as jnp
from jax import lax
from jax.experimental import pallas as pl
from jax.experimental.pallas import tpu as pltpu
from jax.experimental.pallas import tpu_sc as plsc

N = 10000
E = 320000
F = 128
HID = 64
C = 40
K = 10
CP = 64
NW = 32
N2 = 2 * N
NV = N2 // 16
CE = 8000
NCHUNK = E // CE


def _mlp_body(x_ref, w1_ref, b1_ref, w2_ref, b2_ref, o_ref):
    h1 = jnp.maximum(jnp.dot(x_ref[...], w1_ref[...],
                             preferred_element_type=jnp.float32) + b1_ref[...], 0.0)
    o_ref[...] = jnp.dot(h1, w2_ref[...],
                         preferred_element_type=jnp.float32) + b2_ref[...]


def _lsm_body(u_ref, o_ref):
    u = u_ref[...]
    ids = lax.broadcasted_iota(jnp.int32, u.shape, 1)
    m = jnp.where(ids < C, u, jnp.float32(-1e30))
    mx = jnp.max(m, axis=1, keepdims=True)
    e = jnp.exp(m - mx)
    s = jnp.sum(e, axis=1, keepdims=True)
    o_ref[...] = (u - mx - jnp.log(s))[:, :C]


def _sc_body(h_hbm, row_hbm, col_hbm, tb_hbm, out_hbm,
             Vv, Tv, Hqv, dsv, rb0, rb1, cb0, cb1, tv, sem0, sem1):
    wid = lax.axis_index("s") * 2 + lax.axis_index("c")
    ones = jnp.ones((16,), jnp.float32)
    rb = (rb0, rb1)
    cb = (cb0, cb1)
    sems = (sem0, sem1)

    pltpu.sync_copy(tb_hbm, tv)
    pltpu.sync_copy(h_hbm.at[wid], Vv)

    def _issue(ch, b, use_row):
        if use_row:
            pltpu.async_copy(row_hbm.at[pl.ds(ch * CE, CE)], rb[b], sems[b])
        pltpu.async_copy(col_hbm.at[pl.ds(ch * CE, CE)], cb[b], sems[b])

    def _wait(b, use_row):
        if use_row:
            pltpu.make_async_copy(row_hbm.at[pl.ds(0, CE)], rb[b], sems[b]).wait()
        pltpu.make_async_copy(col_hbm.at[pl.ds(0, CE)], cb[b], sems[b]).wait()

    def _edge_pass(grp_fn, use_row):
        def _process(b):
            def _grp(g, cc):
                grp_fn(rb[b], cb[b], g)
                return cc
            lax.fori_loop(0, CE // 16, _grp, 0, unroll=10)

        _issue(0, 0, use_row)

        def _pair(p, c):
            ch0 = p * 2
            _issue(ch0 + 1, 1, use_row)
            _wait(0, use_row)
            _process(0)

            @pl.when(p + 1 < NCHUNK // 2)
            def _():
                _issue(ch0 + 2, 0, use_row)
            _wait(1, use_row)
            _process(1)
            return c
        lax.fori_loop(0, NCHUNK // 2, _pair, 0)

    def _init_t(i, c):
        Tv[pl.ds(i * 16, 16)] = ones
        return c
    lax.fori_loop(0, NV, _init_t, 0, unroll=10)

    def _deg_grp(rbuf, cbuf, g):
        cv = cbuf[pl.ds(g * 16, 16)]
        plsc.addupdate_scatter(Tv, [cv], ones)
        plsc.addupdate_scatter(Tv, [cv + N], ones)
    _edge_pass(_deg_grp, use_row=False)

    t0 = tv[pl.ds(0, 16)]

    def _node_init(i, c):
        sl = pl.ds(i * 16, 16)
        d = Tv[sl]
        di = plsc.bitcast(d, jnp.int32)
        y = plsc.bitcast(jnp.int32(0x5F3759DF) - lax.shift_right_logical(di, 1),
                         jnp.float32)
        for _ in range(4):
            y = y * (jnp.float32(1.5) - jnp.float32(0.5) * d * y * y)
        dsv[sl] = y
        q = y * Vv[sl]
        Vv[sl] = q
        Tv[sl] = q
        Hqv[sl] = t0 * q
        return c
    lax.fori_loop(0, NV, _node_init, 0, unroll=5)

    def _edge_grp(rbuf, cbuf, g):
        sl = pl.ds(g * 16, 16)
        rv = rbuf[sl]
        cv = cbuf[sl]
        g0 = plsc.load_gather(Vv, [rv])
        g1 = plsc.load_gather(Vv, [rv + N])
        plsc.addupdate_scatter(Tv, [cv], g0)
        plsc.addupdate_scatter(Tv, [cv + N], g1)

    for k in range(K):
        _edge_pass(_edge_grp, use_row=True)
        tk = tv[pl.ds(16 * (k + 1), 16)]

        def _scale(i, c):
            sl = pl.ds(i * 16, 16)
            s = dsv[sl]
            q = s * s * Tv[sl]
            Vv[sl] = q
            Tv[sl] = q
            Hqv[sl] = Hqv[sl] + tk * q
            return c
        lax.fori_loop(0, NV, _scale, 0, unroll=10)

    def _fin(i, c):
        sl = pl.ds(i * 16, 16)
        Vv[sl] = Hqv[sl] / dsv[sl]
        return c
    lax.fori_loop(0, NV, _fin, 0, unroll=10)
    pltpu.sync_copy(Vv, out_hbm.at[wid])


_sc_prop = functools.partial(
    pl.kernel,
    out_type=jax.ShapeDtypeStruct((NW, N2), jnp.float32),
    mesh=plsc.VectorSubcoreMesh(core_axis_name="c", subcore_axis_name="s"),
    compiler_params=pltpu.CompilerParams(needs_layout_passes=False),
    scratch_types=[
        pltpu.VMEM((N2,), jnp.float32),
        pltpu.VMEM((N2,), jnp.float32),
        pltpu.VMEM((N2,), jnp.float32),
        pltpu.VMEM((N2,), jnp.float32),
        pltpu.VMEM((CE,), jnp.int32),
        pltpu.VMEM((CE,), jnp.int32),
        pltpu.VMEM((CE,), jnp.int32),
        pltpu.VMEM((CE,), jnp.int32),
        pltpu.VMEM((16 * (K + 1),), jnp.float32),
        pltpu.SemaphoreType.DMA,
        pltpu.SemaphoreType.DMA,
    ],
)(_sc_body)


def kernel(x, edge_index, W1, b1, W2, b2, temp):
    w2p = jnp.pad(W2, ((0, 0), (0, CP - C)))
    b2p = jnp.pad(b2, (0, CP - C)).reshape(1, CP)
    h_pad = pl.pallas_call(
        _mlp_body,
        out_shape=jax.ShapeDtypeStruct((N, CP), jnp.float32),
    )(x, W1, b1.reshape(1, HID), w2p, b2p)

    h32 = h_pad.reshape(N, NW, 2).transpose(1, 2, 0).reshape(NW, N2)
    tb = jnp.broadcast_to(temp.reshape(K + 1, 1), (K + 1, 16)).reshape(-1)
    ei = edge_index.astype(jnp.int32)
    u32 = _sc_prop(h32, ei[0], ei[1], tb)
    u = u32.reshape(NW, 2, N).transpose(2, 0, 1).reshape(N, CP)

    return pl.pallas_call(
        _lsm_body,
        out_shape=jax.ShapeDtypeStruct((N, C), jnp.float32),
    )(u)
